# Initial kernel scaffold; baseline (speedup 1.0000x reference)
import jax
import jax.numpy as jnp
from jax.experimental import pallas as pl


def kernel(indices, table):
    raise NotImplementedError("write your pallas kernel here")


# speedup vs baseline: 6.1305x; 6.1305x over previous
import functools

import jax
import jax.numpy as jnp
from jax import lax
from jax.experimental import pallas as pl
from jax.experimental.pallas import tpu as pltpu
from jax.experimental.pallas import tpu_sc as plsc

EMB_D = 32
IPG = 128
GPC = 8
CHUNK = IPG * GPC


def _sc_gather(table, idx_flat):
    B = idx_flat.shape[0]
    info = plsc.get_sparse_core_info()
    nc, ns = info.num_cores, info.num_subcores
    nw = nc * ns
    b_per_w = B // nw
    assert b_per_w * nw == B
    n_chunks = b_per_w // CHUNK
    assert n_chunks * CHUNK == b_per_w
    rows_per_w = b_per_w // IPG

    idx2 = idx_flat.reshape(B // IPG, IPG)
    mesh = plsc.VectorSubcoreMesh(core_axis_name="c", subcore_axis_name="s")

    @functools.partial(
        pl.kernel,
        mesh=mesh,
        out_type=jax.ShapeDtypeStruct((B, EMB_D), jnp.float32),
        scratch_types=[
            pltpu.VMEM((GPC, IPG), jnp.int32),
            pltpu.VMEM((CHUNK, EMB_D), jnp.float32),
            pltpu.SemaphoreType.DMA,
        ],
        compiler_params=pltpu.CompilerParams(use_tc_tiling_on_sc=False),
    )
    def body(table_hbm, idx_hbm, out_hbm, idx_v, rows_v, sem):
        wid = lax.axis_index("s") * nc + lax.axis_index("c")
        row0 = wid * rows_per_w

        def chunk_body(c, carry):
            r = row0 + c * GPC
            pltpu.sync_copy(idx_hbm.at[pl.ds(r, GPC)], idx_v)
            cps = [
                pltpu.async_copy(
                    table_hbm.at[idx_v.at[j]],
                    rows_v.at[pl.ds(j * IPG, IPG)],
                    sem,
                )
                for j in range(GPC)
            ]
            for cp in cps:
                cp.wait()
            pltpu.sync_copy(rows_v, out_hbm.at[pl.ds(r * IPG, CHUNK)])
            return carry

        lax.fori_loop(0, n_chunks, chunk_body, 0)

    return body(table, idx2)


def kernel(indices, table):
    bt, hist = indices.shape
    flat = indices.reshape(-1).astype(jnp.int32)
    out = _sc_gather(table, flat)
    return out.reshape(bt, hist, table.shape[1])


# speedup vs baseline: 6.3517x; 1.0361x over previous
import functools

import jax
import jax.numpy as jnp
from jax import lax
from jax.experimental import pallas as pl
from jax.experimental.pallas import tpu as pltpu
from jax.experimental.pallas import tpu_sc as plsc

EMB_D = 32
IPG = 128
GPC = 8
CHUNK = IPG * GPC


def _sc_gather(table, idx_flat):
    B = idx_flat.shape[0]
    info = plsc.get_sparse_core_info()
    nc, ns = info.num_cores, info.num_subcores
    nw = nc * ns
    b_per_w = B // nw
    assert b_per_w * nw == B
    n_chunks = b_per_w // CHUNK
    assert n_chunks * CHUNK == b_per_w
    assert n_chunks % 2 == 0
    rows_per_w = b_per_w // IPG

    idx2 = idx_flat.reshape(B // IPG, IPG)
    mesh = plsc.VectorSubcoreMesh(core_axis_name="c", subcore_axis_name="s")

    @functools.partial(
        pl.kernel,
        mesh=mesh,
        out_type=jax.ShapeDtypeStruct((B, EMB_D), jnp.float32),
        scratch_types=[
            pltpu.VMEM((GPC, IPG), jnp.int32),
            pltpu.VMEM((GPC, IPG), jnp.int32),
            pltpu.VMEM((CHUNK, EMB_D), jnp.float32),
            pltpu.VMEM((CHUNK, EMB_D), jnp.float32),
            pltpu.SemaphoreType.DMA,
            pltpu.SemaphoreType.DMA,
            pltpu.SemaphoreType.DMA,
            pltpu.SemaphoreType.DMA,
        ],
        compiler_params=pltpu.CompilerParams(use_tc_tiling_on_sc=False),
    )
    def body(table_hbm, idx_hbm, out_hbm, idx_v0, idx_v1, rows_v0, rows_v1,
             gs0, gs1, os0, os1):
        wid = lax.axis_index("s") * nc + lax.axis_index("c")
        row0 = wid * rows_per_w
        idx_vs = (idx_v0, idx_v1)
        rows_vs = (rows_v0, rows_v1)
        gsems = (gs0, gs1)
        osems = (os0, os1)

        def load_and_fire(c, b):
            r = row0 + c * GPC
            pltpu.sync_copy(idx_hbm.at[pl.ds(r, GPC)], idx_vs[b])
            for j in range(GPC):
                pltpu.async_copy(
                    table_hbm.at[idx_vs[b].at[j]],
                    rows_vs[b].at[pl.ds(j * IPG, IPG)],
                    gsems[b],
                )

        def wait_gathers(c, b):
            r = row0 + c * GPC
            pltpu.make_async_copy(
                out_hbm.at[pl.ds(r * IPG, CHUNK)], rows_vs[b], gsems[b]
            ).wait()

        def start_store(c, b):
            r = row0 + c * GPC
            pltpu.async_copy(
                rows_vs[b], out_hbm.at[pl.ds(r * IPG, CHUNK)], osems[b]
            )

        def wait_store(c, b):
            r = row0 + c * GPC
            pltpu.make_async_copy(
                rows_vs[b], out_hbm.at[pl.ds(r * IPG, CHUNK)], osems[b]
            ).wait()

        def outer(g, carry):
            for b in range(2):
                c = g * 2 + b

                @pl.when(g >= 1)
                def _():
                    wait_store(c - 2, b)

                load_and_fire(c, b)

                if b == 1:
                    wait_gathers(c - 1, 0)
                    start_store(c - 1, 0)
                else:
                    @pl.when(g >= 1)
                    def _():
                        wait_gathers(c - 1, 1)
                        start_store(c - 1, 1)
            return carry

        lax.fori_loop(0, n_chunks // 2, outer, 0)

        last = n_chunks - 1
        wait_gathers(last, last % 2)
        start_store(last, last % 2)
        wait_store(last - 1, (last - 1) % 2)
        wait_store(last, last % 2)

    return body(table, idx2)


def kernel(indices, table):
    bt, hist = indices.shape
    flat = indices.reshape(-1).astype(jnp.int32)
    out = _sc_gather(table, flat)
    return out.reshape(bt, hist, table.shape[1])
